# baseline (speedup 1.0000x reference)
import functools

import jax
import jax.numpy as jnp
from jax import lax
from jax.experimental import pallas as pl
from jax.experimental.pallas import tpu as pltpu
from jax.experimental.pallas import tpu_sc as plsc

V = 1000000
D = 16
DENSE = 13
NF = 26
B = 16384
BNF = B * NF
HB = B // 2
HNF = HB * NF

NC = 2
NS = 16
NW = NC * NS
CHUNK = 128
GROUP = 13


def _sc_mesh():
  return plsc.VectorSubcoreMesh(core_axis_name="c", subcore_axis_name="s")


def _sc_gather_rows(idx2d, emb):
  n_per_w = HNF // NW
  cpw = n_per_w // CHUNK
  groups = cpw // GROUP
  gidx = GROUP * CHUNK

  @functools.partial(
      pl.kernel,
      mesh=_sc_mesh(),
      out_type=jax.ShapeDtypeStruct((HNF, D), jnp.bfloat16),
      scratch_types=[
          pltpu.VMEM((cpw, CHUNK), jnp.int32),
          pltpu.VMEM((gidx, D), jnp.bfloat16),
          pltpu.SemaphoreType.DMA,
      ],
      compiler_params=pltpu.CompilerParams(use_tc_tiling_on_sc=False),
  )
  def k(idx_hbm, emb_hbm, rows_out, idx_v, rows_v, sem):
    wid = lax.axis_index("s") * NC + lax.axis_index("c")
    pltpu.sync_copy(idx_hbm.at[pl.ds(wid * cpw, cpw)], idx_v)

    def body(g, carry):
      fbase = wid * n_per_w + g * gidx
      copies = []
      for j in range(GROUP):
        copies.append(pltpu.async_copy(
            emb_hbm.at[idx_v.at[g * GROUP + j]],
            rows_v.at[pl.ds(j * CHUNK, CHUNK)], sem))
      for c in copies:
        c.wait()
      pltpu.sync_copy(rows_v, rows_out.at[pl.ds(fbase, gidx)])
      return carry

    lax.fori_loop(0, groups, body, 0)

  return k(idx2d, emb)


def _sc_gather_ones(idx2d, emb_one):
  n_per_w = BNF // NW
  cpw = n_per_w // CHUNK
  groups = cpw // GROUP
  gidx = GROUP * CHUNK

  @functools.partial(
      pl.kernel,
      mesh=_sc_mesh(),
      out_type=jax.ShapeDtypeStruct((BNF,), jnp.float32),
      scratch_types=[
          pltpu.VMEM((cpw, CHUNK), jnp.int32),
          pltpu.VMEM((gidx,), jnp.float32),
          pltpu.SemaphoreType.DMA,
      ],
      compiler_params=pltpu.CompilerParams(use_tc_tiling_on_sc=False),
  )
  def k(idx_hbm, one_hbm, ones_out, idx_v, ones_v, sem):
    wid = lax.axis_index("s") * NC + lax.axis_index("c")
    pltpu.sync_copy(idx_hbm.at[pl.ds(wid * cpw, cpw)], idx_v)

    def body(g, carry):
      fbase = wid * n_per_w + g * gidx
      copies = []
      for j in range(GROUP):
        copies.append(pltpu.async_copy(
            one_hbm.at[idx_v.at[g * GROUP + j]],
            ones_v.at[pl.ds(j * CHUNK, CHUNK)], sem))
      for c in copies:
        c.wait()
      pltpu.sync_copy(ones_v, ones_out.at[pl.ds(fbase, gidx)])
      return carry

    lax.fori_loop(0, groups, body, 0)

  return k(idx2d, emb_one)


BM = 1024
SD = NF * D
DD = DENSE * D


def _tc_body(xs_ref, ones_ref, den_ref, wflat_ref, dwone_ref, w0s_ref,
             w0d_ref, b0_ref, w1_ref, b1_ref, w2_ref, b2_ref, w3_ref,
             b3_ref, bias_ref, out_ref):
  f32 = jnp.float32
  bf16 = jnp.bfloat16
  xs_b = xs_ref[...]
  xs = xs_b.astype(f32)
  ones = ones_ref[...]
  den = den_ref[...]
  wflat = wflat_ref[...]

  erow = lax.broadcasted_iota(jnp.int32, (DENSE, DD), 0)
  ecol = lax.broadcasted_iota(jnp.int32, (DENSE, DD), 1)
  expand = (ecol // D == erow).astype(bf16)
  den_b = den.astype(bf16)
  dexp = jnp.dot(den_b, expand, preferred_element_type=f32)
  demb = dexp * wflat

  def selector(n):
    r = lax.broadcasted_iota(jnp.int32, (n, D), 0)
    c = lax.broadcasted_iota(jnp.int32, (n, D), 1)
    return (r % D == c).astype(bf16)

  s1 = selector(SD)
  s2 = selector(DD)
  demb_b = demb.astype(bf16)
  summed = (jnp.dot(xs_b, s1, preferred_element_type=f32)
            + jnp.dot(demb_b, s2, preferred_element_type=f32))
  sq_summed = (jnp.dot((xs * xs).astype(bf16), s1, preferred_element_type=f32)
               + jnp.dot((demb * demb).astype(bf16), s2,
                         preferred_element_type=f32))
  y2 = 0.5 * jnp.sum(summed * summed - sq_summed, axis=1, keepdims=True)

  y1 = (jnp.sum(ones, axis=1, keepdims=True)
        + jnp.dot(den_b, dwone_ref[...], preferred_element_type=f32))

  h = (jnp.dot(xs_b, w0s_ref[...], preferred_element_type=f32)
       + jnp.dot(demb_b, w0d_ref[...], preferred_element_type=f32)
       + b0_ref[...])
  h = jnp.maximum(h, 0.0).astype(bf16)
  h = jnp.dot(h, w1_ref[...], preferred_element_type=f32) + b1_ref[...]
  h = jnp.maximum(h, 0.0).astype(bf16)
  h = jnp.dot(h, w2_ref[...], preferred_element_type=f32) + b2_ref[...]
  h = jnp.maximum(h, 0.0).astype(bf16)
  yd = jnp.dot(h, w3_ref[...], preferred_element_type=f32) + b3_ref[...]

  z = bias_ref[...] + y1 + y2 + yd
  out_ref[...] = jax.nn.sigmoid(z)


def _tc_head(half, xs, ones2d, dense_inputs, wflat, dwone, w0s, w0d, b0,
             w1, b1, w2, b2, w3, b3, bias):
  grid = (HB // BM,)
  off = half * (HB // BM)

  def fullblk(cols):
    return pl.BlockSpec((BM, cols), lambda i: (i + off, 0))

  def full(a):
    return pl.BlockSpec(a.shape, lambda i: (0,) * a.ndim)

  return pl.pallas_call(
      _tc_body,
      grid=grid,
      in_specs=[
          pl.BlockSpec((BM, SD), lambda i: (i, 0)),
          fullblk(NF), fullblk(DENSE),
          full(wflat), full(dwone), full(w0s), full(w0d), full(b0),
          full(w1), full(b1), full(w2), full(b2), full(w3), full(b3),
          full(bias),
      ],
      out_specs=pl.BlockSpec((BM, 1), lambda i: (i, 0)),
      out_shape=jax.ShapeDtypeStruct((HB, 1), jnp.float32),
      compiler_params=pltpu.CompilerParams(
          dimension_semantics=("arbitrary",)),
  )(xs, ones2d, dense_inputs, wflat, dwone, w0s, w0d, b0, w1, b1, w2, b2,
    w3, b3, bias)


def kernel(sparse_inputs, dense_inputs, emb_one, emb, dense_w_one, dense_w,
           W0, b0, W1, b1, W2, b2, W3, b3, bias):
  bf16 = jnp.bfloat16
  idx2d = sparse_inputs.reshape(BNF // CHUNK, CHUNK)
  embh = emb.astype(bf16)
  ones = _sc_gather_ones(idx2d, emb_one.reshape(V))
  ones2d = ones.reshape(B, NF)

  wflat = dense_w.reshape(1, DD)
  dwone = dense_w_one.reshape(DENSE, 1).astype(bf16)
  w0s = W0[:SD].astype(bf16)
  w0d = W0[SD:].astype(bf16)
  wargs = (wflat, dwone, w0s, w0d, b0.reshape(1, -1), W1.astype(bf16),
           b1.reshape(1, -1), W2.astype(bf16), b2.reshape(1, -1),
           W3.astype(bf16), b3.reshape(1, -1), bias.reshape(1, 1))

  hrows = HNF // CHUNK
  outs = []
  for h in range(2):
    rows = _sc_gather_rows(
        lax.slice_in_dim(idx2d, h * hrows, (h + 1) * hrows, axis=0), embh)
    xs = rows.reshape(HB, SD)
    outs.append(_tc_head(h, xs, ones2d, dense_inputs, *wargs))
  return jnp.concatenate(outs, axis=0)


# speedup vs baseline: 1.3788x; 1.3788x over previous
import functools

import jax
import jax.numpy as jnp
from jax import lax
from jax.experimental import pallas as pl
from jax.experimental.pallas import tpu as pltpu
from jax.experimental.pallas import tpu_sc as plsc

V = 1000000
D = 16
DENSE = 13
NF = 26
B = 16384
BNF = B * NF
HB = B // 2
HNF = HB * NF

NC = 2
NS = 16
NW = NC * NS
CHUNK = 128
GROUP = 13


def _sc_mesh():
  return plsc.VectorSubcoreMesh(core_axis_name="c", subcore_axis_name="s")


def _sc_gather_rows(idx2d, emb):
  n_per_w = HNF // NW
  cpw = n_per_w // CHUNK
  groups = cpw // GROUP
  gidx = GROUP * CHUNK

  @functools.partial(
      pl.kernel,
      mesh=_sc_mesh(),
      out_type=jax.ShapeDtypeStruct((HNF, D), jnp.float32),
      scratch_types=[
          pltpu.VMEM((cpw, CHUNK), jnp.int32),
          pltpu.VMEM((gidx, D), jnp.float32),
          pltpu.SemaphoreType.DMA,
      ],
      compiler_params=pltpu.CompilerParams(use_tc_tiling_on_sc=False),
  )
  def k(idx_hbm, emb_hbm, rows_out, idx_v, rows_v, sem):
    wid = lax.axis_index("s") * NC + lax.axis_index("c")
    pltpu.sync_copy(idx_hbm.at[pl.ds(wid * cpw, cpw)], idx_v)

    def body(g, carry):
      fbase = wid * n_per_w + g * gidx
      copies = []
      for j in range(GROUP):
        copies.append(pltpu.async_copy(
            emb_hbm.at[idx_v.at[g * GROUP + j]],
            rows_v.at[pl.ds(j * CHUNK, CHUNK)], sem))
      for c in copies:
        c.wait()
      pltpu.sync_copy(rows_v, rows_out.at[pl.ds(fbase, gidx)])
      return carry

    lax.fori_loop(0, groups, body, 0)

  return k(idx2d, emb)


def _sc_gather_ones(idx2d, emb_one):
  n_per_w = BNF // NW
  cpw = n_per_w // CHUNK
  groups = cpw // GROUP
  gidx = GROUP * CHUNK

  @functools.partial(
      pl.kernel,
      mesh=_sc_mesh(),
      out_type=jax.ShapeDtypeStruct((BNF,), jnp.float32),
      scratch_types=[
          pltpu.VMEM((cpw, CHUNK), jnp.int32),
          pltpu.VMEM((gidx,), jnp.float32),
          pltpu.SemaphoreType.DMA,
      ],
      compiler_params=pltpu.CompilerParams(use_tc_tiling_on_sc=False),
  )
  def k(idx_hbm, one_hbm, ones_out, idx_v, ones_v, sem):
    wid = lax.axis_index("s") * NC + lax.axis_index("c")
    pltpu.sync_copy(idx_hbm.at[pl.ds(wid * cpw, cpw)], idx_v)

    def body(g, carry):
      fbase = wid * n_per_w + g * gidx
      copies = []
      for j in range(GROUP):
        copies.append(pltpu.async_copy(
            one_hbm.at[idx_v.at[g * GROUP + j]],
            ones_v.at[pl.ds(j * CHUNK, CHUNK)], sem))
      for c in copies:
        c.wait()
      pltpu.sync_copy(ones_v, ones_out.at[pl.ds(fbase, gidx)])
      return carry

    lax.fori_loop(0, groups, body, 0)

  return k(idx2d, emb_one)


BM = 1024
SD = NF * D
DD = DENSE * D


def _tc_body(xs_ref, ones_ref, den_ref, wflat_ref, dwone_ref, w0s_ref,
             w0d_ref, b0_ref, w1_ref, b1_ref, w2_ref, b2_ref, w3_ref,
             b3_ref, bias_ref, out_ref):
  f32 = jnp.float32
  bf16 = jnp.bfloat16
  xs = xs_ref[...]
  ones = ones_ref[...]
  den = den_ref[...]
  wflat = wflat_ref[...]

  erow = lax.broadcasted_iota(jnp.int32, (DENSE, DD), 0)
  ecol = lax.broadcasted_iota(jnp.int32, (DENSE, DD), 1)
  expand = (ecol // D == erow).astype(bf16)
  den_b = den.astype(bf16)
  dexp = jnp.dot(den_b, expand, preferred_element_type=f32)
  demb = dexp * wflat

  def selector(n):
    r = lax.broadcasted_iota(jnp.int32, (n, D), 0)
    c = lax.broadcasted_iota(jnp.int32, (n, D), 1)
    return (r % D == c).astype(bf16)

  s1 = selector(SD)
  s2 = selector(DD)
  xs_b = xs.astype(bf16)
  demb_b = demb.astype(bf16)
  summed = (jnp.dot(xs_b, s1, preferred_element_type=f32)
            + jnp.dot(demb_b, s2, preferred_element_type=f32))
  sq_summed = (jnp.dot((xs * xs).astype(bf16), s1, preferred_element_type=f32)
               + jnp.dot((demb * demb).astype(bf16), s2,
                         preferred_element_type=f32))
  y2 = 0.5 * jnp.sum(summed * summed - sq_summed, axis=1, keepdims=True)

  y1 = (jnp.sum(ones, axis=1, keepdims=True)
        + jnp.dot(den_b, dwone_ref[...], preferred_element_type=f32))

  h = (jnp.dot(xs_b, w0s_ref[...], preferred_element_type=f32)
       + jnp.dot(demb_b, w0d_ref[...], preferred_element_type=f32)
       + b0_ref[...])
  h = jnp.maximum(h, 0.0).astype(bf16)
  h = jnp.dot(h, w1_ref[...], preferred_element_type=f32) + b1_ref[...]
  h = jnp.maximum(h, 0.0).astype(bf16)
  h = jnp.dot(h, w2_ref[...], preferred_element_type=f32) + b2_ref[...]
  h = jnp.maximum(h, 0.0).astype(bf16)
  yd = jnp.dot(h, w3_ref[...], preferred_element_type=f32) + b3_ref[...]

  z = bias_ref[...] + y1 + y2 + yd
  out_ref[...] = jax.nn.sigmoid(z)


def _tc_head(half, xs, ones2d, dense_inputs, wflat, dwone, w0s, w0d, b0,
             w1, b1, w2, b2, w3, b3, bias):
  grid = (HB // BM,)
  off = half * (HB // BM)

  def fullblk(cols):
    return pl.BlockSpec((BM, cols), lambda i: (i + off, 0))

  def full(a):
    return pl.BlockSpec(a.shape, lambda i: (0,) * a.ndim)

  return pl.pallas_call(
      _tc_body,
      grid=grid,
      in_specs=[
          pl.BlockSpec((BM, SD), lambda i: (i, 0)),
          fullblk(NF), fullblk(DENSE),
          full(wflat), full(dwone), full(w0s), full(w0d), full(b0),
          full(w1), full(b1), full(w2), full(b2), full(w3), full(b3),
          full(bias),
      ],
      out_specs=pl.BlockSpec((BM, 1), lambda i: (i, 0)),
      out_shape=jax.ShapeDtypeStruct((HB, 1), jnp.float32),
      compiler_params=pltpu.CompilerParams(
          dimension_semantics=("arbitrary",)),
  )(xs, ones2d, dense_inputs, wflat, dwone, w0s, w0d, b0, w1, b1, w2, b2,
    w3, b3, bias)


def kernel(sparse_inputs, dense_inputs, emb_one, emb, dense_w_one, dense_w,
           W0, b0, W1, b1, W2, b2, W3, b3, bias):
  bf16 = jnp.bfloat16
  idx2d = sparse_inputs.reshape(BNF // CHUNK, CHUNK)
  ones = _sc_gather_ones(idx2d, emb_one.reshape(V))
  ones2d = ones.reshape(B, NF)

  wflat = dense_w.reshape(1, DD)
  dwone = dense_w_one.reshape(DENSE, 1).astype(bf16)
  w0s = W0[:SD].astype(bf16)
  w0d = W0[SD:].astype(bf16)
  wargs = (wflat, dwone, w0s, w0d, b0.reshape(1, -1), W1.astype(bf16),
           b1.reshape(1, -1), W2.astype(bf16), b2.reshape(1, -1),
           W3.astype(bf16), b3.reshape(1, -1), bias.reshape(1, 1))

  hrows = HNF // CHUNK
  outs = []
  for h in range(2):
    rows = _sc_gather_rows(
        lax.slice_in_dim(idx2d, h * hrows, (h + 1) * hrows, axis=0), emb)
    xs = rows.reshape(HB, SD)
    outs.append(_tc_head(h, xs, ones2d, dense_inputs, *wargs))
  return jnp.concatenate(outs, axis=0)


# speedup vs baseline: 1.3789x; 1.0001x over previous
import functools

import jax
import jax.numpy as jnp
from jax import lax
from jax.experimental import pallas as pl
from jax.experimental.pallas import tpu as pltpu
from jax.experimental.pallas import tpu_sc as plsc

V = 1000000
D = 16
DENSE = 13
NF = 26
B = 16384
BNF = B * NF
HB = B // 2
HNF = HB * NF

NC = 2
NS = 16
NW = NC * NS
CHUNK = 128
GROUP = 13


def _sc_mesh():
  return plsc.VectorSubcoreMesh(core_axis_name="c", subcore_axis_name="s")


def _sc_gather_rows(idx2d, emb):
  n_per_w = HNF // NW
  cpw = n_per_w // CHUNK
  groups = cpw // GROUP
  gidx = GROUP * CHUNK

  @functools.partial(
      pl.kernel,
      mesh=_sc_mesh(),
      out_type=jax.ShapeDtypeStruct((HNF, D), jnp.float32),
      scratch_types=[
          pltpu.VMEM((cpw, CHUNK), jnp.int32),
          pltpu.VMEM((gidx, D), jnp.float32),
          pltpu.SemaphoreType.DMA,
      ],
      compiler_params=pltpu.CompilerParams(use_tc_tiling_on_sc=False),
  )
  def k(idx_hbm, emb_hbm, rows_out, idx_v, rows_v, sem):
    wid = lax.axis_index("s") * NC + lax.axis_index("c")
    pltpu.sync_copy(idx_hbm.at[pl.ds(wid * cpw, cpw)], idx_v)

    def body(g, carry):
      fbase = wid * n_per_w + g * gidx
      copies = []
      for j in range(GROUP):
        copies.append(pltpu.async_copy(
            emb_hbm.at[idx_v.at[g * GROUP + j]],
            rows_v.at[pl.ds(j * CHUNK, CHUNK)], sem))
      for c in copies:
        c.wait()
      pltpu.sync_copy(rows_v, rows_out.at[pl.ds(fbase, gidx)])
      return carry

    lax.fori_loop(0, groups, body, 0)

  return k(idx2d, emb)


def _sc_gather_ones(idx2d, emb_one):
  n_per_w = BNF // NW
  cpw = n_per_w // CHUNK
  groups = cpw // GROUP
  gidx = GROUP * CHUNK

  @functools.partial(
      pl.kernel,
      mesh=_sc_mesh(),
      out_type=jax.ShapeDtypeStruct((BNF,), jnp.float32),
      scratch_types=[
          pltpu.VMEM((cpw, CHUNK), jnp.int32),
          pltpu.VMEM((gidx,), jnp.float32),
          pltpu.SemaphoreType.DMA,
      ],
      compiler_params=pltpu.CompilerParams(use_tc_tiling_on_sc=False),
  )
  def k(idx_hbm, one_hbm, ones_out, idx_v, ones_v, sem):
    wid = lax.axis_index("s") * NC + lax.axis_index("c")
    pltpu.sync_copy(idx_hbm.at[pl.ds(wid * cpw, cpw)], idx_v)

    def body(g, carry):
      fbase = wid * n_per_w + g * gidx
      copies = []
      for j in range(GROUP):
        copies.append(pltpu.async_copy(
            one_hbm.at[idx_v.at[g * GROUP + j]],
            ones_v.at[pl.ds(j * CHUNK, CHUNK)], sem))
      for c in copies:
        c.wait()
      pltpu.sync_copy(ones_v, ones_out.at[pl.ds(fbase, gidx)])
      return carry

    lax.fori_loop(0, groups, body, 0)

  return k(idx2d, emb_one)


BM = 1024
SD = NF * D
DD = DENSE * D


def _tc_body(xs_ref, ones_ref, den_ref, wflat_ref, dwone_ref, w0s_ref,
             w0d_ref, b0_ref, w1_ref, b1_ref, w2_ref, b2_ref, w3_ref,
             b3_ref, bias_ref, out_ref):
  f32 = jnp.float32
  bf16 = jnp.bfloat16
  xs = xs_ref[...]
  ones = ones_ref[...]
  den = den_ref[...]
  wflat = wflat_ref[...]

  erow = lax.broadcasted_iota(jnp.int32, (DENSE, DD), 0)
  ecol = lax.broadcasted_iota(jnp.int32, (DENSE, DD), 1)
  expand = (ecol // D == erow).astype(bf16)
  den_b = den.astype(bf16)
  dexp = jnp.dot(den_b, expand, preferred_element_type=f32)
  demb = dexp * wflat

  def selector(n):
    r = lax.broadcasted_iota(jnp.int32, (n, D), 0)
    c = lax.broadcasted_iota(jnp.int32, (n, D), 1)
    return (r % D == c).astype(bf16)

  s1 = selector(SD)
  s2 = selector(DD)
  xs_b = xs.astype(bf16)
  demb_b = demb.astype(bf16)
  summed = (jnp.dot(xs_b, s1, preferred_element_type=f32)
            + jnp.dot(demb_b, s2, preferred_element_type=f32))
  sq_summed = (jnp.dot((xs * xs).astype(bf16), s1, preferred_element_type=f32)
               + jnp.dot((demb * demb).astype(bf16), s2,
                         preferred_element_type=f32))
  y2 = 0.5 * jnp.sum(summed * summed - sq_summed, axis=1, keepdims=True)

  y1 = (jnp.sum(ones, axis=1, keepdims=True)
        + jnp.dot(den_b, dwone_ref[...], preferred_element_type=f32))

  h = (jnp.dot(xs_b, w0s_ref[...], preferred_element_type=f32)
       + jnp.dot(demb_b, w0d_ref[...], preferred_element_type=f32)
       + b0_ref[...])
  h = jnp.maximum(h, 0.0).astype(bf16)
  h = jnp.dot(h, w1_ref[...], preferred_element_type=f32) + b1_ref[...]
  h = jnp.maximum(h, 0.0).astype(bf16)
  h = jnp.dot(h, w2_ref[...], preferred_element_type=f32) + b2_ref[...]
  h = jnp.maximum(h, 0.0).astype(bf16)
  yd = jnp.dot(h, w3_ref[...], preferred_element_type=f32) + b3_ref[...]

  z = bias_ref[...] + y1 + y2 + yd
  out_ref[...] = jax.nn.sigmoid(z)


def _tc_head(half, xs, ones2d, dense_inputs, wflat, dwone, w0s, w0d, b0,
             w1, b1, w2, b2, w3, b3, bias):
  grid = (HB // BM,)
  off = half * (HB // BM)

  def fullblk(cols):
    return pl.BlockSpec((BM, cols), lambda i: (i + off, 0))

  def full(a):
    return pl.BlockSpec(a.shape, lambda i: (0,) * a.ndim)

  return pl.pallas_call(
      _tc_body,
      grid=grid,
      in_specs=[
          pl.BlockSpec((BM, SD), lambda i: (i, 0)),
          fullblk(NF), fullblk(DENSE),
          full(wflat), full(dwone), full(w0s), full(w0d), full(b0),
          full(w1), full(b1), full(w2), full(b2), full(w3), full(b3),
          full(bias),
      ],
      out_specs=pl.BlockSpec((BM, 1), lambda i: (i, 0)),
      out_shape=jax.ShapeDtypeStruct((HB, 1), jnp.float32),
      compiler_params=pltpu.CompilerParams(
          dimension_semantics=("parallel",)),
  )(xs, ones2d, dense_inputs, wflat, dwone, w0s, w0d, b0, w1, b1, w2, b2,
    w3, b3, bias)


def kernel(sparse_inputs, dense_inputs, emb_one, emb, dense_w_one, dense_w,
           W0, b0, W1, b1, W2, b2, W3, b3, bias):
  bf16 = jnp.bfloat16
  idx2d = sparse_inputs.reshape(BNF // CHUNK, CHUNK)
  ones = _sc_gather_ones(idx2d, emb_one.reshape(V))
  ones2d = ones.reshape(B, NF)

  wflat = dense_w.reshape(1, DD)
  dwone = dense_w_one.reshape(DENSE, 1).astype(bf16)
  w0s = W0[:SD].astype(bf16)
  w0d = W0[SD:].astype(bf16)
  wargs = (wflat, dwone, w0s, w0d, b0.reshape(1, -1), W1.astype(bf16),
           b1.reshape(1, -1), W2.astype(bf16), b2.reshape(1, -1),
           W3.astype(bf16), b3.reshape(1, -1), bias.reshape(1, 1))

  hrows = HNF // CHUNK
  outs = []
  for h in range(2):
    rows = _sc_gather_rows(
        lax.slice_in_dim(idx2d, h * hrows, (h + 1) * hrows, axis=0), emb)
    xs = rows.reshape(HB, SD)
    outs.append(_tc_head(h, xs, ones2d, dense_inputs, *wargs))
  return jnp.concatenate(outs, axis=0)
